# Initial kernel scaffold; baseline (speedup 1.0000x reference)
import jax
import jax.numpy as jnp
from jax.experimental import pallas as pl


def kernel(x, edge_index, W1, b1, W2, b2):
    raise NotImplementedError("write your pallas kernel here")


# speedup vs baseline: 23.3563x; 23.3563x over previous
import functools

import jax
import jax.numpy as jnp
from jax import lax
from jax.experimental import pallas as pl
from jax.experimental.pallas import tpu as pltpu
from jax.experimental.pallas import tpu_sc as plsc

N_NODES = 10000
N_EDGES = 160000
D_IN = 256
D_HID = 16
D_OUT = 256

NUM_TILES = 32
CH = 128
NCH = 40
E_PAD = NUM_TILES * NCH * CH
N_PAD = 10112
ROWS_PER_TILE = N_PAD // 16

_sc_mesh = plsc.VectorSubcoreMesh(core_axis_name="c", subcore_axis_name="s")


@functools.partial(
    pl.kernel,
    out_type=jax.ShapeDtypeStruct((2, N_PAD, D_HID), jnp.float32),
    mesh=_sc_mesh,
    scratch_types=[
        pltpu.VMEM((NCH, CH), jnp.int32),
        pltpu.VMEM((CH, D_HID), jnp.float32),
        pltpu.VMEM_SHARED((N_PAD, D_HID), jnp.float32),
    ],
    compiler_params=pltpu.CompilerParams(use_tc_tiling_on_sc=False),
)
def _sc_deg(cols_hbm, ones_hbm, zeros_hbm, out_hbm, cidx, ones_v, acc):
    c = lax.axis_index("c")
    s = lax.axis_index("s")
    wid = c * 16 + s
    base = s * ROWS_PER_TILE
    pltpu.sync_copy(zeros_hbm.at[pl.ds(base, ROWS_PER_TILE)],
                    acc.at[pl.ds(base, ROWS_PER_TILE)])
    pltpu.sync_copy(cols_hbm.at[wid], cidx)
    pltpu.sync_copy(ones_hbm, ones_v)
    plsc.subcore_barrier()

    def body(j, carry):
        pltpu.sync_copy(ones_v, acc.at[cidx.at[j]], add=True)
        return carry

    lax.fori_loop(0, NCH, body, 0)
    plsc.subcore_barrier()
    pltpu.sync_copy(acc.at[pl.ds(base, ROWS_PER_TILE)],
                    out_hbm.at[c].at[pl.ds(base, ROWS_PER_TILE)])


@functools.partial(
    pl.kernel,
    out_type=jax.ShapeDtypeStruct((2, N_PAD, D_HID), jnp.float32),
    mesh=_sc_mesh,
    scratch_types=[
        pltpu.VMEM((NCH, CH), jnp.int32),
        pltpu.VMEM((NCH, CH), jnp.int32),
        pltpu.VMEM((CH, D_HID), jnp.float32),
        pltpu.VMEM_SHARED((N_PAD, D_HID), jnp.float32),
        pltpu.SemaphoreType.DMA,
    ],
    compiler_params=pltpu.CompilerParams(use_tc_tiling_on_sc=False),
)
def _sc_agg(rows_hbm, cols_hbm, table_hbm, zeros_hbm, out_hbm,
            ridx, cidx, buf, acc, sem):
    c = lax.axis_index("c")
    s = lax.axis_index("s")
    wid = c * 16 + s
    base = s * ROWS_PER_TILE
    pltpu.sync_copy(zeros_hbm.at[pl.ds(base, ROWS_PER_TILE)],
                    acc.at[pl.ds(base, ROWS_PER_TILE)])
    pltpu.sync_copy(rows_hbm.at[wid], ridx)
    pltpu.sync_copy(cols_hbm.at[wid], cidx)
    plsc.subcore_barrier()

    def body(j, carry):
        pltpu.async_copy(table_hbm.at[ridx.at[j]], buf, sem).wait()
        pltpu.sync_copy(buf, acc.at[cidx.at[j]], add=True)
        return carry

    lax.fori_loop(0, NCH, body, 0)
    plsc.subcore_barrier()
    pltpu.sync_copy(acc.at[pl.ds(base, ROWS_PER_TILE)],
                    out_hbm.at[c].at[pl.ds(base, ROWS_PER_TILE)])


_BM = 2000


def _mm1_body(x_ref, w_ref, o_ref):
    o_ref[...] = jnp.dot(x_ref[...], w_ref[...],
                         preferred_element_type=jnp.float32)


def _tc_mm1(x, W1):
    return pl.pallas_call(
        _mm1_body,
        grid=(N_NODES // _BM,),
        in_specs=[
            pl.BlockSpec((_BM, D_IN), lambda i: (i, 0)),
            pl.BlockSpec((D_IN, D_HID), lambda i: (0, 0)),
        ],
        out_specs=pl.BlockSpec((_BM, D_HID), lambda i: (i, 0)),
        out_shape=jax.ShapeDtypeStruct((N_NODES, D_HID), jnp.float32),
    )(x, W1)


def _scale_body(x1_ref, d0_ref, d1_ref, xs_ref, dinv_ref):
    deg = d0_ref[...] + d1_ref[...] + 1.0
    dinv = lax.rsqrt(deg)
    dinv_ref[...] = dinv
    xs_ref[...] = x1_ref[...] * dinv


def _tc_scale(X1, d0, d1):
    spec = pl.BlockSpec((_BM, D_HID), lambda i: (i, 0))
    return pl.pallas_call(
        _scale_body,
        grid=(N_NODES // _BM,),
        in_specs=[spec, spec, spec],
        out_specs=[spec, spec],
        out_shape=[
            jax.ShapeDtypeStruct((N_NODES, D_HID), jnp.float32),
            jax.ShapeDtypeStruct((N_NODES, D_HID), jnp.float32),
        ],
    )(X1, d0, d1)


def _mid_body(a0_ref, a1_ref, dinv_ref, x1_ref, b1_ref, h_ref, hs_ref):
    dinv = dinv_ref[...]
    agg = dinv * (a0_ref[...] + a1_ref[...]) + dinv * dinv * x1_ref[...]
    h = jnp.maximum(agg + b1_ref[...], 0.0)
    h_ref[...] = h
    hs_ref[...] = h * dinv


def _tc_mid(a0, a1, dinv, X1, b1):
    spec = pl.BlockSpec((_BM, D_HID), lambda i: (i, 0))
    return pl.pallas_call(
        _mid_body,
        grid=(N_NODES // _BM,),
        in_specs=[spec, spec, spec, spec,
                  pl.BlockSpec((1, D_HID), lambda i: (0, 0))],
        out_specs=[spec, spec],
        out_shape=[
            jax.ShapeDtypeStruct((N_NODES, D_HID), jnp.float32),
            jax.ShapeDtypeStruct((N_NODES, D_HID), jnp.float32),
        ],
    )(a0, a1, dinv, X1, b1)


def _out_body(a0_ref, a1_ref, dinv_ref, h_ref, w2_ref, b2_ref, o_ref):
    dinv = dinv_ref[...]
    agg = dinv * (a0_ref[...] + a1_ref[...]) + dinv * dinv * h_ref[...]
    o_ref[...] = jnp.dot(agg, w2_ref[...],
                         preferred_element_type=jnp.float32) + b2_ref[...]


def _tc_out(a0, a1, dinv, h, W2, b2):
    spec = pl.BlockSpec((_BM, D_HID), lambda i: (i, 0))
    return pl.pallas_call(
        _out_body,
        grid=(N_NODES // _BM,),
        in_specs=[spec, spec, spec, spec,
                  pl.BlockSpec((D_HID, D_OUT), lambda i: (0, 0)),
                  pl.BlockSpec((1, D_OUT), lambda i: (0, 0))],
        out_specs=pl.BlockSpec((_BM, D_OUT), lambda i: (i, 0)),
        out_shape=jax.ShapeDtypeStruct((N_NODES, D_OUT), jnp.float32),
    )(a0, a1, dinv, h, W2, b2)


def kernel(x, edge_index, W1, b1, W2, b2):
    ei = edge_index.astype(jnp.int32)
    pad = E_PAD - N_EDGES
    rows = jnp.concatenate([ei[0], jnp.zeros((pad,), jnp.int32)])
    cols = jnp.concatenate([ei[1], jnp.full((pad,), N_NODES, jnp.int32)])
    rows = rows.reshape(NUM_TILES, NCH, CH)
    cols = cols.reshape(NUM_TILES, NCH, CH)
    zeros_big = jnp.zeros((N_PAD, D_HID), jnp.float32)
    ones_small = jnp.ones((CH, D_HID), jnp.float32)

    degp = _sc_deg(cols, ones_small, zeros_big)
    X1 = _tc_mm1(x, W1)
    X1s, dinv = _tc_scale(X1, degp[0, :N_NODES], degp[1, :N_NODES])

    aggp1 = _sc_agg(rows, cols, X1s, zeros_big)
    h, hs = _tc_mid(aggp1[0, :N_NODES], aggp1[1, :N_NODES], dinv, X1,
                    b1.reshape(1, D_HID).astype(jnp.float32))

    aggp2 = _sc_agg(rows, cols, hs, zeros_big)
    return _tc_out(aggp2[0, :N_NODES], aggp2[1, :N_NODES], dinv, h,
                   W2, b2.reshape(1, D_OUT).astype(jnp.float32))


# speedup vs baseline: 28.2596x; 1.2099x over previous
import functools

import jax
import jax.numpy as jnp
from jax import lax
from jax.experimental import pallas as pl
from jax.experimental.pallas import tpu as pltpu
from jax.experimental.pallas import tpu_sc as plsc

N_NODES = 10000
N_EDGES = 160000
D_IN = 256
D_HID = 16
D_OUT = 256

NUM_TILES = 32
CH = 128
NCH = 40
E_PAD = NUM_TILES * NCH * CH
N_PAD = 10112
ROWS_PER_TILE = N_PAD // 16

_sc_mesh = plsc.VectorSubcoreMesh(core_axis_name="c", subcore_axis_name="s")


@functools.partial(
    pl.kernel,
    out_type=jax.ShapeDtypeStruct((2, N_PAD, D_HID), jnp.float32),
    mesh=_sc_mesh,
    scratch_types=[
        pltpu.VMEM((NCH, CH), jnp.int32),
        pltpu.VMEM((CH, D_HID), jnp.float32),
        pltpu.VMEM_SHARED((N_PAD, D_HID), jnp.float32),
    ],
    compiler_params=pltpu.CompilerParams(use_tc_tiling_on_sc=False),
)
def _sc_deg(cols_hbm, ones_hbm, zeros_hbm, out_hbm, cidx, ones_v, acc):
    c = lax.axis_index("c")
    s = lax.axis_index("s")
    wid = c * 16 + s
    base = s * ROWS_PER_TILE
    pltpu.sync_copy(zeros_hbm.at[pl.ds(base, ROWS_PER_TILE)],
                    acc.at[pl.ds(base, ROWS_PER_TILE)])
    pltpu.sync_copy(cols_hbm.at[wid], cidx)
    pltpu.sync_copy(ones_hbm, ones_v)
    plsc.subcore_barrier()

    def body(j, carry):
        pltpu.sync_copy(ones_v, acc.at[cidx.at[j]], add=True)
        return carry

    lax.fori_loop(0, NCH, body, 0)
    plsc.subcore_barrier()
    pltpu.sync_copy(acc.at[pl.ds(base, ROWS_PER_TILE)],
                    out_hbm.at[c].at[pl.ds(base, ROWS_PER_TILE)])


@functools.partial(
    pl.kernel,
    out_type=jax.ShapeDtypeStruct((2, N_PAD, D_HID), jnp.float32),
    mesh=_sc_mesh,
    scratch_types=[
        pltpu.VMEM((NCH, CH), jnp.int32),
        pltpu.VMEM((NCH, CH), jnp.int32),
        pltpu.VMEM((CH, D_HID), jnp.float32),
        pltpu.VMEM((CH, D_HID), jnp.float32),
        pltpu.VMEM((CH, D_HID), jnp.float32),
        pltpu.VMEM((CH, D_HID), jnp.float32),
        pltpu.VMEM_SHARED((N_PAD, D_HID), jnp.float32),
        pltpu.SemaphoreType.DMA,
        pltpu.SemaphoreType.DMA,
        pltpu.SemaphoreType.DMA,
        pltpu.SemaphoreType.DMA,
        pltpu.SemaphoreType.DMA,
        pltpu.SemaphoreType.DMA,
        pltpu.SemaphoreType.DMA,
        pltpu.SemaphoreType.DMA,
    ],
    compiler_params=pltpu.CompilerParams(use_tc_tiling_on_sc=False),
)
def _sc_agg(rows_hbm, cols_hbm, table_hbm, zeros_hbm, out_hbm,
            ridx, cidx, b0, b1, b2, b3,
            acc, g0, g1, g2, g3, s0, s1, s2, s3):
    c = lax.axis_index("c")
    s = lax.axis_index("s")
    wid = c * 16 + s
    base = s * ROWS_PER_TILE
    gb = [b0, b1, b2, b3]
    gs = [g0, g1, g2, g3]
    ss = [s0, s1, s2, s3]
    NB = 4
    pltpu.sync_copy(zeros_hbm.at[pl.ds(base, ROWS_PER_TILE)],
                    acc.at[pl.ds(base, ROWS_PER_TILE)])
    pltpu.sync_copy(rows_hbm.at[wid], ridx)
    pltpu.sync_copy(cols_hbm.at[wid], cidx)
    plsc.subcore_barrier()

    for b in range(NB):
        pltpu.async_copy(table_hbm.at[ridx.at[b]], gb[b], gs[b])

    def outer(it, carry):
        j0 = it * NB
        for b in range(NB):
            j = j0 + b
            pltpu.make_async_copy(table_hbm.at[ridx.at[j]], gb[b],
                                  gs[b]).wait()
            pltpu.async_copy(gb[b], acc.at[cidx.at[j]], ss[b], add=True)
            pltpu.make_async_copy(gb[b], acc.at[cidx.at[j]], ss[b]).wait()

            @pl.when(j + NB < NCH)
            def _():
                pltpu.async_copy(table_hbm.at[ridx.at[j + NB]], gb[b], gs[b])
        return carry

    lax.fori_loop(0, NCH // NB, outer, 0)
    plsc.subcore_barrier()
    pltpu.sync_copy(acc.at[pl.ds(base, ROWS_PER_TILE)],
                    out_hbm.at[c].at[pl.ds(base, ROWS_PER_TILE)])


_BM = 2000


def _mm1_body(x_ref, w_ref, o_ref):
    o_ref[...] = jnp.dot(x_ref[...], w_ref[...],
                         preferred_element_type=jnp.float32)


def _tc_mm1(x, W1):
    return pl.pallas_call(
        _mm1_body,
        grid=(N_NODES // _BM,),
        in_specs=[
            pl.BlockSpec((_BM, D_IN), lambda i: (i, 0)),
            pl.BlockSpec((D_IN, D_HID), lambda i: (0, 0)),
        ],
        out_specs=pl.BlockSpec((_BM, D_HID), lambda i: (i, 0)),
        out_shape=jax.ShapeDtypeStruct((N_NODES, D_HID), jnp.float32),
    )(x, W1)


def _scale_body(x1_ref, d0_ref, d1_ref, xs_ref, dinv_ref):
    deg = d0_ref[...] + d1_ref[...] + 1.0
    dinv = lax.rsqrt(deg)
    dinv_ref[...] = dinv
    xs_ref[...] = x1_ref[...] * dinv


def _tc_scale(X1, d0, d1):
    spec = pl.BlockSpec((_BM, D_HID), lambda i: (i, 0))
    return pl.pallas_call(
        _scale_body,
        grid=(N_NODES // _BM,),
        in_specs=[spec, spec, spec],
        out_specs=[spec, spec],
        out_shape=[
            jax.ShapeDtypeStruct((N_NODES, D_HID), jnp.float32),
            jax.ShapeDtypeStruct((N_NODES, D_HID), jnp.float32),
        ],
    )(X1, d0, d1)


def _mid_body(a0_ref, a1_ref, dinv_ref, x1_ref, b1_ref, h_ref, hs_ref):
    dinv = dinv_ref[...]
    agg = dinv * (a0_ref[...] + a1_ref[...]) + dinv * dinv * x1_ref[...]
    h = jnp.maximum(agg + b1_ref[...], 0.0)
    h_ref[...] = h
    hs_ref[...] = h * dinv


def _tc_mid(a0, a1, dinv, X1, b1):
    spec = pl.BlockSpec((_BM, D_HID), lambda i: (i, 0))
    return pl.pallas_call(
        _mid_body,
        grid=(N_NODES // _BM,),
        in_specs=[spec, spec, spec, spec,
                  pl.BlockSpec((1, D_HID), lambda i: (0, 0))],
        out_specs=[spec, spec],
        out_shape=[
            jax.ShapeDtypeStruct((N_NODES, D_HID), jnp.float32),
            jax.ShapeDtypeStruct((N_NODES, D_HID), jnp.float32),
        ],
    )(a0, a1, dinv, X1, b1)


def _out_body(a0_ref, a1_ref, dinv_ref, h_ref, w2_ref, b2_ref, o_ref):
    dinv = dinv_ref[...]
    agg = dinv * (a0_ref[...] + a1_ref[...]) + dinv * dinv * h_ref[...]
    o_ref[...] = jnp.dot(agg, w2_ref[...],
                         preferred_element_type=jnp.float32) + b2_ref[...]


def _tc_out(a0, a1, dinv, h, W2, b2):
    spec = pl.BlockSpec((_BM, D_HID), lambda i: (i, 0))
    return pl.pallas_call(
        _out_body,
        grid=(N_NODES // _BM,),
        in_specs=[spec, spec, spec, spec,
                  pl.BlockSpec((D_HID, D_OUT), lambda i: (0, 0)),
                  pl.BlockSpec((1, D_OUT), lambda i: (0, 0))],
        out_specs=pl.BlockSpec((_BM, D_OUT), lambda i: (i, 0)),
        out_shape=jax.ShapeDtypeStruct((N_NODES, D_OUT), jnp.float32),
    )(a0, a1, dinv, h, W2, b2)


def kernel(x, edge_index, W1, b1, W2, b2):
    ei = edge_index.astype(jnp.int32)
    pad = E_PAD - N_EDGES
    rows = jnp.concatenate([ei[0], jnp.zeros((pad,), jnp.int32)])
    cols = jnp.concatenate([ei[1], jnp.full((pad,), N_NODES, jnp.int32)])
    rows = rows.reshape(NUM_TILES, NCH, CH)
    cols = cols.reshape(NUM_TILES, NCH, CH)
    zeros_big = jnp.zeros((N_PAD, D_HID), jnp.float32)
    ones_small = jnp.ones((CH, D_HID), jnp.float32)

    degp = _sc_deg(cols, ones_small, zeros_big)
    X1 = _tc_mm1(x, W1)
    X1s, dinv = _tc_scale(X1, degp[0, :N_NODES], degp[1, :N_NODES])

    aggp1 = _sc_agg(rows, cols, X1s, zeros_big)
    h, hs = _tc_mid(aggp1[0, :N_NODES], aggp1[1, :N_NODES], dinv, X1,
                    b1.reshape(1, D_HID).astype(jnp.float32))

    aggp2 = _sc_agg(rows, cols, hs, zeros_big)
    return _tc_out(aggp2[0, :N_NODES], aggp2[1, :N_NODES], dinv, h,
                   W2, b2.reshape(1, D_OUT).astype(jnp.float32))


# speedup vs baseline: 34.8999x; 1.2350x over previous
import functools

import jax
import jax.numpy as jnp
from jax import lax
from jax.experimental import pallas as pl
from jax.experimental.pallas import tpu as pltpu
from jax.experimental.pallas import tpu_sc as plsc

N_NODES = 10000
N_EDGES = 160000
D_IN = 256
D_HID = 16
D_OUT = 256

NUM_TILES = 32
CH = 128
NCH = 40
NB = 4
E_PAD = NUM_TILES * NCH * CH
N_PAD = 10240
RPT16 = N_PAD // 16
RPT32 = N_PAD // 32

_sc_mesh = plsc.VectorSubcoreMesh(core_axis_name="c", subcore_axis_name="s")
_sc_params = pltpu.CompilerParams(use_tc_tiling_on_sc=False,
                                  needs_layout_passes=False)


@functools.partial(
    pl.kernel,
    out_type=jax.ShapeDtypeStruct((2, N_PAD, D_HID), jnp.float32),
    mesh=_sc_mesh,
    scratch_types=[
        pltpu.VMEM((NCH, CH), jnp.int32),
        pltpu.VMEM((CH, D_HID), jnp.float32),
        pltpu.VMEM_SHARED((N_PAD, D_HID), jnp.float32),
    ],
    compiler_params=_sc_params,
)
def _sc_deg(cols_hbm, ones_hbm, zeros_hbm, out_hbm, cidx, ones_v, acc):
    c = lax.axis_index("c")
    s = lax.axis_index("s")
    wid = c * 16 + s
    base = s * RPT16
    pltpu.sync_copy(zeros_hbm.at[pl.ds(base, RPT16)],
                    acc.at[pl.ds(base, RPT16)])
    pltpu.sync_copy(cols_hbm.at[wid], cidx)
    pltpu.sync_copy(ones_hbm, ones_v)
    plsc.subcore_barrier()

    def body(j, carry):
        pltpu.sync_copy(ones_v, acc.at[cidx.at[j]], add=True)
        return carry

    lax.fori_loop(0, NCH, body, 0)
    plsc.subcore_barrier()
    pltpu.sync_copy(acc.at[pl.ds(base, RPT16)],
                    out_hbm.at[c].at[pl.ds(base, RPT16)])


@functools.partial(
    pl.kernel,
    out_type=jax.ShapeDtypeStruct((2, N_PAD, D_HID), jnp.float32),
    mesh=_sc_mesh,
    scratch_types=[
        pltpu.VMEM((NCH, CH), jnp.int32),
        pltpu.VMEM((NCH, CH), jnp.int32),
        pltpu.VMEM((CH, D_HID), jnp.float32),
        pltpu.VMEM((CH, D_HID), jnp.float32),
        pltpu.VMEM((CH, D_HID), jnp.float32),
        pltpu.VMEM((CH, D_HID), jnp.float32),
        pltpu.VMEM_SHARED((N_PAD, D_HID), jnp.float32),
        pltpu.SemaphoreType.DMA,
        pltpu.SemaphoreType.DMA,
        pltpu.SemaphoreType.DMA,
        pltpu.SemaphoreType.DMA,
        pltpu.SemaphoreType.DMA,
        pltpu.SemaphoreType.DMA,
        pltpu.SemaphoreType.DMA,
        pltpu.SemaphoreType.DMA,
    ],
    compiler_params=_sc_params,
)
def _sc_agg(rows_hbm, cols_hbm, table_hbm, zeros_hbm, out_hbm,
            ridx, cidx, b0, b1, b2, b3,
            acc, g0, g1, g2, g3, s0, s1, s2, s3):
    c = lax.axis_index("c")
    s = lax.axis_index("s")
    wid = c * 16 + s
    base = s * RPT16
    gb = [b0, b1, b2, b3]
    gs = [g0, g1, g2, g3]
    ss = [s0, s1, s2, s3]
    pltpu.sync_copy(zeros_hbm.at[pl.ds(base, RPT16)],
                    acc.at[pl.ds(base, RPT16)])
    pltpu.sync_copy(rows_hbm.at[wid], ridx)
    pltpu.sync_copy(cols_hbm.at[wid], cidx)
    plsc.subcore_barrier()

    for b in range(NB):
        pltpu.async_copy(table_hbm.at[ridx.at[b]], gb[b], gs[b])

    def outer(it, carry):
        j0 = it * NB
        for b in range(NB):
            j = j0 + b
            pltpu.make_async_copy(table_hbm.at[ridx.at[j]], gb[b],
                                  gs[b]).wait()
            pltpu.async_copy(gb[b], acc.at[cidx.at[j]], ss[b], add=True)
            pltpu.make_async_copy(gb[b], acc.at[cidx.at[j]], ss[b]).wait()

            @pl.when(j + NB < NCH)
            def _():
                pltpu.async_copy(table_hbm.at[ridx.at[j + NB]], gb[b], gs[b])
        return carry

    lax.fori_loop(0, NCH // NB, outer, 0)
    plsc.subcore_barrier()
    pltpu.sync_copy(acc.at[pl.ds(base, RPT16)],
                    out_hbm.at[c].at[pl.ds(base, RPT16)])


def _bit_rsqrt(deg):
    i = plsc.bitcast(deg, jnp.int32)
    y = plsc.bitcast(jnp.int32(0x5F3759DF) - (i >> 1), jnp.float32)
    y = y * (1.5 - 0.5 * deg * y * y)
    y = y * (1.5 - 0.5 * deg * y * y)
    y = y * (1.5 - 0.5 * deg * y * y)
    return y


@functools.partial(
    pl.kernel,
    out_type=[jax.ShapeDtypeStruct((N_PAD, D_HID), jnp.float32),
              jax.ShapeDtypeStruct((N_PAD, D_HID), jnp.float32)],
    mesh=_sc_mesh,
    scratch_types=[
        pltpu.VMEM((RPT32, D_HID), jnp.float32),
        pltpu.VMEM((RPT32, D_HID), jnp.float32),
        pltpu.VMEM((RPT32, D_HID), jnp.float32),
        pltpu.VMEM((RPT32, D_HID), jnp.float32),
        pltpu.VMEM((RPT32, D_HID), jnp.float32),
    ],
    compiler_params=_sc_params,
)
def _sc_prep(degp_hbm, x1_hbm, xs_hbm, dv_hbm, d0b, d1b, x1b, xsb, dvb):
    c = lax.axis_index("c")
    s = lax.axis_index("s")
    base = (c * 16 + s) * RPT32
    pltpu.sync_copy(degp_hbm.at[0].at[pl.ds(base, RPT32)], d0b)
    pltpu.sync_copy(degp_hbm.at[1].at[pl.ds(base, RPT32)], d1b)
    pltpu.sync_copy(x1_hbm.at[pl.ds(base, RPT32)], x1b)

    def body(r, carry):
        y = _bit_rsqrt(d0b[r] + d1b[r] + 1.0)
        dvb[r] = y
        xsb[r] = x1b[r] * y
        return carry

    lax.fori_loop(0, RPT32, body, 0)
    pltpu.sync_copy(xsb, xs_hbm.at[pl.ds(base, RPT32)])
    pltpu.sync_copy(dvb, dv_hbm.at[pl.ds(base, RPT32)])


@functools.partial(
    pl.kernel,
    out_type=jax.ShapeDtypeStruct((N_PAD, D_HID), jnp.float32),
    mesh=_sc_mesh,
    scratch_types=[
        pltpu.VMEM((RPT32, D_HID), jnp.float32),
        pltpu.VMEM((RPT32, D_HID), jnp.float32),
        pltpu.VMEM((RPT32, D_HID), jnp.float32),
        pltpu.VMEM((RPT32, D_HID), jnp.float32),
        pltpu.VMEM((RPT32, D_HID), jnp.float32),
        pltpu.VMEM((D_HID,), jnp.float32),
    ],
    compiler_params=_sc_params,
)
def _sc_mid(p_hbm, dv_hbm, xs_hbm, b1_hbm, hs_hbm,
            p0b, p1b, dvb, xsb, hsb, b1v):
    c = lax.axis_index("c")
    s = lax.axis_index("s")
    base = (c * 16 + s) * RPT32
    pltpu.sync_copy(p_hbm.at[0].at[pl.ds(base, RPT32)], p0b)
    pltpu.sync_copy(p_hbm.at[1].at[pl.ds(base, RPT32)], p1b)
    pltpu.sync_copy(dv_hbm.at[pl.ds(base, RPT32)], dvb)
    pltpu.sync_copy(xs_hbm.at[pl.ds(base, RPT32)], xsb)
    pltpu.sync_copy(b1_hbm, b1v)

    def body(r, carry):
        h = jnp.maximum(dvb[r] * (p0b[r] + p1b[r] + xsb[r]) + b1v[...], 0.0)
        hsb[r] = h * dvb[r]
        return carry

    lax.fori_loop(0, RPT32, body, 0)
    pltpu.sync_copy(hsb, hs_hbm.at[pl.ds(base, RPT32)])


@functools.partial(
    pl.kernel,
    out_type=jax.ShapeDtypeStruct((N_PAD, D_HID), jnp.float32),
    mesh=_sc_mesh,
    scratch_types=[
        pltpu.VMEM((RPT32, D_HID), jnp.float32),
        pltpu.VMEM((RPT32, D_HID), jnp.float32),
        pltpu.VMEM((RPT32, D_HID), jnp.float32),
        pltpu.VMEM((RPT32, D_HID), jnp.float32),
        pltpu.VMEM((RPT32, D_HID), jnp.float32),
    ],
    compiler_params=_sc_params,
)
def _sc_fin(p_hbm, dv_hbm, hs_hbm, g_hbm, p0b, p1b, dvb, hsb, gb):
    c = lax.axis_index("c")
    s = lax.axis_index("s")
    base = (c * 16 + s) * RPT32
    pltpu.sync_copy(p_hbm.at[0].at[pl.ds(base, RPT32)], p0b)
    pltpu.sync_copy(p_hbm.at[1].at[pl.ds(base, RPT32)], p1b)
    pltpu.sync_copy(dv_hbm.at[pl.ds(base, RPT32)], dvb)
    pltpu.sync_copy(hs_hbm.at[pl.ds(base, RPT32)], hsb)

    def body(r, carry):
        gb[r] = dvb[r] * (p0b[r] + p1b[r] + hsb[r])
        return carry

    lax.fori_loop(0, RPT32, body, 0)
    pltpu.sync_copy(gb, g_hbm.at[pl.ds(base, RPT32)])


_BM1 = RPT16


def _mm1_body(x_ref, w_ref, o_ref):
    o_ref[...] = jnp.dot(x_ref[...], w_ref[...],
                         preferred_element_type=jnp.float32)


def _tc_mm1(x, W1):
    return pl.pallas_call(
        _mm1_body,
        grid=(N_PAD // _BM1,),
        in_specs=[
            pl.BlockSpec((_BM1, D_IN), lambda i: (i, 0)),
            pl.BlockSpec((D_IN, D_HID), lambda i: (0, 0)),
        ],
        out_specs=pl.BlockSpec((_BM1, D_HID), lambda i: (i, 0)),
        out_shape=jax.ShapeDtypeStruct((N_PAD, D_HID), jnp.float32),
    )(x, W1)


_BM2 = 2000


def _out_body(g_ref, w2_ref, b2_ref, o_ref):
    o_ref[...] = jnp.dot(g_ref[...], w2_ref[...],
                         preferred_element_type=jnp.float32) + b2_ref[...]


def _tc_out(g, W2, b2):
    return pl.pallas_call(
        _out_body,
        grid=(N_NODES // _BM2,),
        in_specs=[pl.BlockSpec((_BM2, D_HID), lambda i: (i, 0)),
                  pl.BlockSpec((D_HID, D_OUT), lambda i: (0, 0)),
                  pl.BlockSpec((1, D_OUT), lambda i: (0, 0))],
        out_specs=pl.BlockSpec((_BM2, D_OUT), lambda i: (i, 0)),
        out_shape=jax.ShapeDtypeStruct((N_NODES, D_OUT), jnp.float32),
    )(g, W2, b2)


def kernel(x, edge_index, W1, b1, W2, b2):
    ei = edge_index.astype(jnp.int32)
    pad = E_PAD - N_EDGES
    rows = jnp.concatenate([ei[0], jnp.zeros((pad,), jnp.int32)])
    cols = jnp.concatenate([ei[1], jnp.full((pad,), N_NODES, jnp.int32)])
    rows = rows.reshape(NUM_TILES, NCH, CH)
    cols = cols.reshape(NUM_TILES, NCH, CH)
    zeros_big = jnp.zeros((N_PAD, D_HID), jnp.float32)
    ones_small = jnp.ones((CH, D_HID), jnp.float32)

    degp = _sc_deg(cols, ones_small, zeros_big)
    X1 = _tc_mm1(x, W1)
    xs, dv = _sc_prep(degp, X1)

    p1 = _sc_agg(rows, cols, xs, zeros_big)
    hs = _sc_mid(p1, dv, xs, b1.astype(jnp.float32))

    p2 = _sc_agg(rows, cols, hs, zeros_big)
    g = _sc_fin(p2, dv, hs)
    return _tc_out(g, W2, b2.reshape(1, D_OUT).astype(jnp.float32))


# speedup vs baseline: 45.5045x; 1.3039x over previous
import functools

import jax
import jax.numpy as jnp
from jax import lax
from jax.experimental import pallas as pl
from jax.experimental.pallas import tpu as pltpu
from jax.experimental.pallas import tpu_sc as plsc

N_NODES = 10000
N_EDGES = 160000
D_IN = 256
D_HID = 16
D_OUT = 256

NUM_TILES = 32
CH = 125
NCH = 40
NB = 8
SKEW = 4
N_PAD = 10240
RPT16 = N_PAD // 16
RPT32 = N_PAD // 32

_sc_mesh = plsc.VectorSubcoreMesh(core_axis_name="c", subcore_axis_name="s")
_sc_params = pltpu.CompilerParams(use_tc_tiling_on_sc=False,
                                  needs_layout_passes=False)


@functools.partial(
    pl.kernel,
    out_type=jax.ShapeDtypeStruct((2, N_PAD, D_HID), jnp.float32),
    mesh=_sc_mesh,
    scratch_types=[
        pltpu.VMEM((NCH, CH), jnp.int32),
        pltpu.VMEM((CH, D_HID), jnp.float32),
        pltpu.VMEM_SHARED((N_PAD, D_HID), jnp.float32),
        pltpu.SemaphoreType.DMA,
    ],
    compiler_params=_sc_params,
)
def _sc_deg(cols_hbm, ones_hbm, zeros_hbm, out_hbm, cidx, ones_v, acc, dsem):
    c = lax.axis_index("c")
    s = lax.axis_index("s")
    wid = c * 16 + s
    base = s * RPT16
    pltpu.sync_copy(zeros_hbm.at[pl.ds(base, RPT16)],
                    acc.at[pl.ds(base, RPT16)])
    pltpu.sync_copy(cols_hbm.at[wid], cidx)
    pltpu.sync_copy(ones_hbm, ones_v)
    plsc.subcore_barrier()

    def fire(j, carry):
        pltpu.async_copy(ones_v, acc.at[cidx.at[j]], dsem, add=True)
        return carry

    def drain(j, carry):
        pltpu.make_async_copy(ones_v, acc.at[cidx.at[j]], dsem).wait()
        return carry

    lax.fori_loop(0, NCH, fire, 0)
    lax.fori_loop(0, NCH, drain, 0)
    plsc.subcore_barrier()
    pltpu.sync_copy(acc.at[pl.ds(base, RPT16)],
                    out_hbm.at[c].at[pl.ds(base, RPT16)])


@functools.partial(
    pl.kernel,
    out_type=jax.ShapeDtypeStruct((2, N_PAD, D_HID), jnp.float32),
    mesh=_sc_mesh,
    scratch_types=[
        pltpu.VMEM((NCH, CH), jnp.int32),
        pltpu.VMEM((NCH, CH), jnp.int32),
        pltpu.VMEM((CH, D_HID), jnp.float32),
        pltpu.VMEM((CH, D_HID), jnp.float32),
        pltpu.VMEM((CH, D_HID), jnp.float32),
        pltpu.VMEM((CH, D_HID), jnp.float32),
        pltpu.VMEM((CH, D_HID), jnp.float32),
        pltpu.VMEM((CH, D_HID), jnp.float32),
        pltpu.VMEM((CH, D_HID), jnp.float32),
        pltpu.VMEM((CH, D_HID), jnp.float32),
        pltpu.VMEM_SHARED((N_PAD, D_HID), jnp.float32),
        pltpu.SemaphoreType.DMA,
        pltpu.SemaphoreType.DMA,
        pltpu.SemaphoreType.DMA,
        pltpu.SemaphoreType.DMA,
        pltpu.SemaphoreType.DMA,
        pltpu.SemaphoreType.DMA,
        pltpu.SemaphoreType.DMA,
        pltpu.SemaphoreType.DMA,
        pltpu.SemaphoreType.DMA,
        pltpu.SemaphoreType.DMA,
        pltpu.SemaphoreType.DMA,
        pltpu.SemaphoreType.DMA,
        pltpu.SemaphoreType.DMA,
        pltpu.SemaphoreType.DMA,
        pltpu.SemaphoreType.DMA,
        pltpu.SemaphoreType.DMA,
    ],
    compiler_params=_sc_params,
)
def _sc_agg(rows_hbm, cols_hbm, table_hbm, zeros_hbm, out_hbm,
            ridx, cidx, b0, b1, b2, b3, b4, b5, b6, b7,
            acc, g0, g1, g2, g3, g4, g5, g6, g7,
            s0, s1, s2, s3, s4, s5, s6, s7):
    c = lax.axis_index("c")
    s = lax.axis_index("s")
    wid = c * 16 + s
    base = s * RPT16
    gb = [b0, b1, b2, b3, b4, b5, b6, b7]
    gs = [g0, g1, g2, g3, g4, g5, g6, g7]
    ss = [s0, s1, s2, s3, s4, s5, s6, s7]
    pltpu.sync_copy(zeros_hbm.at[pl.ds(base, RPT16)],
                    acc.at[pl.ds(base, RPT16)])
    pltpu.sync_copy(rows_hbm.at[wid], ridx)
    pltpu.sync_copy(cols_hbm.at[wid], cidx)
    plsc.subcore_barrier()

    for b in range(SKEW):
        pltpu.async_copy(table_hbm.at[ridx.at[b]], gb[b], gs[b])

    def outer(it, carry):
        j0 = it * NB
        for b in range(NB):
            j = j0 + b
            bn = (b + SKEW) % NB
            pltpu.make_async_copy(table_hbm.at[ridx.at[j]], gb[b],
                                  gs[b]).wait()
            pltpu.async_copy(gb[b], acc.at[cidx.at[j]], ss[b], add=True)

            @pl.when(j >= SKEW)
            def _():
                pltpu.make_async_copy(gb[bn], acc.at[cidx.at[j - SKEW]],
                                      ss[bn]).wait()

            @pl.when(j + SKEW < NCH)
            def _():
                pltpu.async_copy(table_hbm.at[ridx.at[j + SKEW]],
                                 gb[bn], gs[bn])
        return carry

    lax.fori_loop(0, NCH // NB, outer, 0)
    for k in range(SKEW):
        j = NCH - SKEW + k
        b = j % NB
        pltpu.make_async_copy(gb[b], acc.at[cidx.at[j]], ss[b]).wait()
    plsc.subcore_barrier()
    pltpu.sync_copy(acc.at[pl.ds(base, RPT16)],
                    out_hbm.at[c].at[pl.ds(base, RPT16)])


def _bit_rsqrt(deg):
    i = plsc.bitcast(deg, jnp.int32)
    y = plsc.bitcast(jnp.int32(0x5F3759DF) - (i >> 1), jnp.float32)
    y = y * (1.5 - 0.5 * deg * y * y)
    y = y * (1.5 - 0.5 * deg * y * y)
    y = y * (1.5 - 0.5 * deg * y * y)
    return y


@functools.partial(
    pl.kernel,
    out_type=[jax.ShapeDtypeStruct((N_PAD, D_HID), jnp.float32),
              jax.ShapeDtypeStruct((N_PAD, D_HID), jnp.float32)],
    mesh=_sc_mesh,
    scratch_types=[
        pltpu.VMEM((RPT32, D_HID), jnp.float32),
        pltpu.VMEM((RPT32, D_HID), jnp.float32),
        pltpu.VMEM((RPT32, D_HID), jnp.float32),
        pltpu.VMEM((RPT32, D_HID), jnp.float32),
        pltpu.VMEM((RPT32, D_HID), jnp.float32),
    ],
    compiler_params=_sc_params,
)
def _sc_prep(degp_hbm, x1_hbm, xs_hbm, dv_hbm, d0b, d1b, x1b, xsb, dvb):
    c = lax.axis_index("c")
    s = lax.axis_index("s")
    base = (c * 16 + s) * RPT32
    pltpu.sync_copy(degp_hbm.at[0].at[pl.ds(base, RPT32)], d0b)
    pltpu.sync_copy(degp_hbm.at[1].at[pl.ds(base, RPT32)], d1b)
    pltpu.sync_copy(x1_hbm.at[pl.ds(base, RPT32)], x1b)

    def body(r, carry):
        y = _bit_rsqrt(d0b[r] + d1b[r] + 1.0)
        dvb[r] = y
        xsb[r] = x1b[r] * y
        return carry

    lax.fori_loop(0, RPT32, body, 0)
    pltpu.sync_copy(xsb, xs_hbm.at[pl.ds(base, RPT32)])
    pltpu.sync_copy(dvb, dv_hbm.at[pl.ds(base, RPT32)])


@functools.partial(
    pl.kernel,
    out_type=jax.ShapeDtypeStruct((N_PAD, D_HID), jnp.float32),
    mesh=_sc_mesh,
    scratch_types=[
        pltpu.VMEM((RPT32, D_HID), jnp.float32),
        pltpu.VMEM((RPT32, D_HID), jnp.float32),
        pltpu.VMEM((RPT32, D_HID), jnp.float32),
        pltpu.VMEM((RPT32, D_HID), jnp.float32),
        pltpu.VMEM((RPT32, D_HID), jnp.float32),
        pltpu.VMEM((D_HID,), jnp.float32),
    ],
    compiler_params=_sc_params,
)
def _sc_mid(p_hbm, dv_hbm, xs_hbm, b1_hbm, hs_hbm,
            p0b, p1b, dvb, xsb, hsb, b1v):
    c = lax.axis_index("c")
    s = lax.axis_index("s")
    base = (c * 16 + s) * RPT32
    pltpu.sync_copy(p_hbm.at[0].at[pl.ds(base, RPT32)], p0b)
    pltpu.sync_copy(p_hbm.at[1].at[pl.ds(base, RPT32)], p1b)
    pltpu.sync_copy(dv_hbm.at[pl.ds(base, RPT32)], dvb)
    pltpu.sync_copy(xs_hbm.at[pl.ds(base, RPT32)], xsb)
    pltpu.sync_copy(b1_hbm, b1v)

    def body(r, carry):
        h = jnp.maximum(dvb[r] * (p0b[r] + p1b[r] + xsb[r]) + b1v[...], 0.0)
        hsb[r] = h * dvb[r]
        return carry

    lax.fori_loop(0, RPT32, body, 0)
    pltpu.sync_copy(hsb, hs_hbm.at[pl.ds(base, RPT32)])


@functools.partial(
    pl.kernel,
    out_type=jax.ShapeDtypeStruct((N_PAD, D_HID), jnp.float32),
    mesh=_sc_mesh,
    scratch_types=[
        pltpu.VMEM((RPT32, D_HID), jnp.float32),
        pltpu.VMEM((RPT32, D_HID), jnp.float32),
        pltpu.VMEM((RPT32, D_HID), jnp.float32),
        pltpu.VMEM((RPT32, D_HID), jnp.float32),
        pltpu.VMEM((RPT32, D_HID), jnp.float32),
    ],
    compiler_params=_sc_params,
)
def _sc_fin(p_hbm, dv_hbm, hs_hbm, g_hbm, p0b, p1b, dvb, hsb, gb):
    c = lax.axis_index("c")
    s = lax.axis_index("s")
    base = (c * 16 + s) * RPT32
    pltpu.sync_copy(p_hbm.at[0].at[pl.ds(base, RPT32)], p0b)
    pltpu.sync_copy(p_hbm.at[1].at[pl.ds(base, RPT32)], p1b)
    pltpu.sync_copy(dv_hbm.at[pl.ds(base, RPT32)], dvb)
    pltpu.sync_copy(hs_hbm.at[pl.ds(base, RPT32)], hsb)

    def body(r, carry):
        gb[r] = dvb[r] * (p0b[r] + p1b[r] + hsb[r])
        return carry

    lax.fori_loop(0, RPT32, body, 0)
    pltpu.sync_copy(gb, g_hbm.at[pl.ds(base, RPT32)])


_BM1 = 2048


def _mm1_body(x_ref, w_ref, o_ref):
    o_ref[...] = jnp.dot(x_ref[...], w_ref[...],
                         preferred_element_type=jnp.float32)


def _tc_mm1(x, W1):
    return pl.pallas_call(
        _mm1_body,
        grid=(N_PAD // _BM1,),
        in_specs=[
            pl.BlockSpec((_BM1, D_IN), lambda i: (i, 0)),
            pl.BlockSpec((D_IN, D_HID), lambda i: (0, 0)),
        ],
        out_specs=pl.BlockSpec((_BM1, D_HID), lambda i: (i, 0)),
        out_shape=jax.ShapeDtypeStruct((N_PAD, D_HID), jnp.float32),
    )(x, W1)


_BM2 = 2000


def _out_body(g_ref, w2_ref, b2_ref, o_ref):
    o_ref[...] = jnp.dot(g_ref[...], w2_ref[...],
                         preferred_element_type=jnp.float32) + b2_ref[...]


def _tc_out(g, W2, b2):
    return pl.pallas_call(
        _out_body,
        grid=(N_NODES // _BM2,),
        in_specs=[pl.BlockSpec((_BM2, D_HID), lambda i: (i, 0)),
                  pl.BlockSpec((D_HID, D_OUT), lambda i: (0, 0)),
                  pl.BlockSpec((1, D_OUT), lambda i: (0, 0))],
        out_specs=pl.BlockSpec((_BM2, D_OUT), lambda i: (i, 0)),
        out_shape=jax.ShapeDtypeStruct((N_NODES, D_OUT), jnp.float32),
    )(g, W2, b2)


def kernel(x, edge_index, W1, b1, W2, b2):
    ei = edge_index.astype(jnp.int32)
    rows = ei[0].reshape(NUM_TILES, NCH, CH)
    cols = ei[1].reshape(NUM_TILES, NCH, CH)
    zeros_big = jnp.zeros((N_PAD, D_HID), jnp.float32)
    ones_small = jnp.ones((CH, D_HID), jnp.float32)

    degp = _sc_deg(cols, ones_small, zeros_big)
    X1 = _tc_mm1(x, W1)
    xs, dv = _sc_prep(degp, X1)

    p1 = _sc_agg(rows, cols, xs, zeros_big)
    hs = _sc_mid(p1, dv, xs, b1.astype(jnp.float32))

    p2 = _sc_agg(rows, cols, hs, zeros_big)
    g = _sc_fin(p2, dv, hs)
    return _tc_out(g, W2, b2.reshape(1, D_OUT).astype(jnp.float32))
